# Initial kernel scaffold; baseline (speedup 1.0000x reference)
import jax
import jax.numpy as jnp
from jax.experimental import pallas as pl


def kernel(x, W_enc, W_dec):
    raise NotImplementedError("write your pallas kernel here")


# speedup vs baseline: 2.4758x; 2.4758x over previous
import functools

import jax
import jax.numpy as jnp
from jax import lax
from jax.experimental import pallas as pl
from jax.experimental.pallas import tpu as pltpu

K_TOP = 64
MIN32 = -(2 ** 31)
POS_MASK = 0x7FFFFFFF


def _f32_key(v):
    b = lax.bitcast_convert_type(v, jnp.int32)
    return jnp.where(b >= 0, b, b ^ POS_MASK)


def _enc_topk_kernel(x_ref, w_ref, out_ref, keys_ref, *, hb, nh, bt, hidden,
                     chunk):
    j = pl.program_id(1)
    enc = lax.dot_general(
        x_ref[...], w_ref[...], (((1,), (1,)), ((), ())),
        preferred_element_type=jnp.float32)
    out_ref[:, pl.ds(j * hb, hb)] = enc
    keys_ref[:, pl.ds(j * hb, hb)] = _f32_key(enc)

    @pl.when(j == nh - 1)
    def _finalize():
        nchunks = hidden // chunk

        def count_ge(t_signed):
            def chunk_body(c, cnt):
                kc = keys_ref[:, pl.ds(c * chunk, chunk)]
                return cnt + jnp.sum((kc >= t_signed).astype(jnp.int32),
                                     axis=1, keepdims=True)
            return lax.fori_loop(0, nchunks, chunk_body,
                                 jnp.zeros((bt, 1), jnp.int32))

        def bit_body(t, prefix):
            bitval = lax.shift_left(jnp.int32(1), 31 - t)
            cand = prefix | bitval
            cnt = count_ge(cand ^ MIN32)
            return jnp.where(cnt >= K_TOP, cand, prefix)

        prefix = lax.fori_loop(0, 32, bit_body, jnp.zeros((bt, 1), jnp.int32))
        t_signed = prefix ^ MIN32

        def mask_body(c, carry):
            sl = pl.ds(c * chunk, chunk)
            kc = keys_ref[:, sl]
            out_ref[:, sl] = jnp.where(kc >= t_signed, out_ref[:, sl], 0.0)
            return carry

        lax.fori_loop(0, nchunks, mask_body, 0)


def _dec_kernel(s_ref, w_ref, out_ref):
    j = pl.program_id(1)
    prod = lax.dot_general(
        s_ref[...], w_ref[...], (((1,), (1,)), ((), ())),
        preferred_element_type=jnp.float32)

    @pl.when(j == 0)
    def _init():
        out_ref[...] = prod

    @pl.when(j != 0)
    def _acc():
        out_ref[...] += prod


@jax.jit
def kernel(x, W_enc, W_dec):
    batch, din = x.shape
    hidden = W_enc.shape[0]

    bt = min(128, batch)
    hb = min(1024, hidden)
    nb, nh = batch // bt, hidden // hb
    chunk = min(512, hidden)

    sparse = pl.pallas_call(
        functools.partial(_enc_topk_kernel, hb=hb, nh=nh, bt=bt,
                          hidden=hidden, chunk=chunk),
        grid=(nb, nh),
        in_specs=[
            pl.BlockSpec((bt, din), lambda i, j: (i, 0)),
            pl.BlockSpec((hb, din), lambda i, j: (j, 0)),
        ],
        out_specs=pl.BlockSpec((bt, hidden), lambda i, j: (i, 0)),
        out_shape=jax.ShapeDtypeStruct((batch, hidden), jnp.float32),
        scratch_shapes=[pltpu.VMEM((bt, hidden), jnp.int32)],
        compiler_params=pltpu.CompilerParams(
            vmem_limit_bytes=110 * 1024 * 1024),
    )(x, W_enc)

    bt2 = min(256, batch)
    hb2 = min(1024, hidden)
    decoded = pl.pallas_call(
        _dec_kernel,
        grid=(batch // bt2, hidden // hb2),
        in_specs=[
            pl.BlockSpec((bt2, hb2), lambda i, j: (i, j)),
            pl.BlockSpec((din, hb2), lambda i, j: (0, j)),
        ],
        out_specs=pl.BlockSpec((bt2, din), lambda i, j: (i, 0)),
        out_shape=jax.ShapeDtypeStruct((batch, din), jnp.float32),
        compiler_params=pltpu.CompilerParams(
            vmem_limit_bytes=110 * 1024 * 1024),
    )(sparse, W_dec)

    return (decoded, sparse)


# speedup vs baseline: 5.0854x; 2.0540x over previous
import functools

import jax
import jax.numpy as jnp
from jax import lax
from jax.experimental import pallas as pl
from jax.experimental.pallas import tpu as pltpu

K_TOP = 64
MIN32 = -(2 ** 31)
POS_MASK = 0x7FFFFFFF


def _key_of_bits(b):
    return jnp.where(b >= 0, b, b ^ POS_MASK)


def _enc_topk_kernel(x_ref, w_ref, out_ref, scr_ref, sem, *, hb, nh, nb, bt,
                     hidden, chunk):
    i = pl.program_id(0)
    j = pl.program_id(1)

    @pl.when((j == 0) & (i > 0))
    def _wait_prev():
        pltpu.make_async_copy(
            scr_ref, out_ref.at[pl.ds((i - 1) * bt, bt), :], sem).wait()

    enc = lax.dot_general(
        x_ref[...], w_ref[...], (((1,), (1,)), ((), ())),
        preferred_element_type=jnp.float32)
    k = _key_of_bits(lax.bitcast_convert_type(enc, jnp.int32))
    scr_ref[:, pl.ds(j * hb, hb)] = lax.bitcast_convert_type(k, jnp.float32)

    @pl.when(j == nh - 1)
    def _finalize():
        nchunks = hidden // chunk

        def count_ge(t_signed):
            def chunk_body(c, cnt):
                kc = lax.bitcast_convert_type(
                    scr_ref[:, pl.ds(c * chunk, chunk)], jnp.int32)
                return cnt + jnp.sum((kc >= t_signed).astype(jnp.int32),
                                     axis=1, keepdims=True)
            return lax.fori_loop(0, nchunks, chunk_body,
                                 jnp.zeros((bt, 1), jnp.int32))

        def bit_body(t, prefix):
            bitval = lax.shift_left(jnp.int32(1), 31 - t)
            cand = prefix | bitval
            cnt = count_ge(cand ^ MIN32)
            return jnp.where(cnt >= K_TOP, cand, prefix)

        prefix = lax.fori_loop(0, 32, bit_body, jnp.zeros((bt, 1), jnp.int32))
        t_signed = prefix ^ MIN32

        def mask_body(c, carry):
            sl = pl.ds(c * chunk, chunk)
            kc = lax.bitcast_convert_type(scr_ref[:, sl], jnp.int32)
            val = lax.bitcast_convert_type(_key_of_bits(kc), jnp.float32)
            scr_ref[:, sl] = jnp.where(kc >= t_signed, val, 0.0)
            return carry

        lax.fori_loop(0, nchunks, mask_body, 0)

        copy = pltpu.make_async_copy(
            scr_ref, out_ref.at[pl.ds(i * bt, bt), :], sem)
        copy.start()

        @pl.when(i == nb - 1)
        def _wait_last():
            copy.wait()


def _dec_kernel(s_ref, w_ref, out_ref, acc_ref, sem, *, nh, nb, bt):
    i = pl.program_id(0)
    j = pl.program_id(1)

    @pl.when((j == 0) & (i > 0))
    def _wait_prev():
        pltpu.make_async_copy(
            acc_ref, out_ref.at[pl.ds((i - 1) * bt, bt), :], sem).wait()

    prod = lax.dot_general(
        s_ref[...].astype(jnp.bfloat16), w_ref[...],
        (((1,), (1,)), ((), ())), preferred_element_type=jnp.float32)

    @pl.when(j == 0)
    def _init():
        acc_ref[...] = prod

    @pl.when(j != 0)
    def _acc():
        acc_ref[...] += prod

    @pl.when(j == nh - 1)
    def _flush():
        copy = pltpu.make_async_copy(
            acc_ref, out_ref.at[pl.ds(i * bt, bt), :], sem)
        copy.start()

        @pl.when(i == nb - 1)
        def _wait_last():
            copy.wait()


@jax.jit
def kernel(x, W_enc, W_dec):
    batch, din = x.shape
    hidden = W_enc.shape[0]

    bt = min(512, batch)
    hb = min(256, hidden)
    nb, nh = batch // bt, hidden // hb
    chunk = min(512, hidden)

    sparse = pl.pallas_call(
        functools.partial(_enc_topk_kernel, hb=hb, nh=nh, nb=nb, bt=bt,
                          hidden=hidden, chunk=chunk),
        grid=(nb, nh),
        in_specs=[
            pl.BlockSpec((bt, din), lambda i, j: (i, 0)),
            pl.BlockSpec((hb, din), lambda i, j: (j, 0)),
        ],
        out_specs=pl.BlockSpec(memory_space=pltpu.MemorySpace.HBM),
        out_shape=jax.ShapeDtypeStruct((batch, hidden), jnp.float32),
        scratch_shapes=[pltpu.VMEM((bt, hidden), jnp.float32),
                        pltpu.SemaphoreType.DMA],
        compiler_params=pltpu.CompilerParams(
            vmem_limit_bytes=63 * 1024 * 1024),
    )(x, W_enc)

    bt2 = min(1024, batch)
    hb2 = min(512, hidden)
    nb2, nh2 = batch // bt2, hidden // hb2
    decoded = pl.pallas_call(
        functools.partial(_dec_kernel, nh=nh2, nb=nb2, bt=bt2),
        grid=(nb2, nh2),
        in_specs=[
            pl.BlockSpec((bt2, hb2), lambda i, j: (i, j)),
            pl.BlockSpec((din, hb2), lambda i, j: (0, j)),
        ],
        out_specs=pl.BlockSpec(memory_space=pltpu.MemorySpace.HBM),
        out_shape=jax.ShapeDtypeStruct((batch, din), jnp.float32),
        scratch_shapes=[pltpu.VMEM((bt2, din), jnp.float32),
                        pltpu.SemaphoreType.DMA],
        compiler_params=pltpu.CompilerParams(
            vmem_limit_bytes=63 * 1024 * 1024),
    )(sparse, W_dec.astype(jnp.bfloat16))

    return (decoded, sparse)
